# baseline (speedup 1.0000x reference)
import functools

import jax
import jax.numpy as jnp
from jax.experimental import pallas as pl
from jax.experimental.pallas import tpu as pltpu

_LANE_BLK = 8192
_NSPLIT = 5


def _mlp_kernel(x0, x1, x2, x3, x4, w1_ref, b1_ref, w2_ref, b2_ref, w3_ref,
                b3_ref, o_ref):
    xs = (x0, x1, x2, x3, x4)
    w1 = w1_ref[...]
    ks = w1.shape[1] // _NSPLIT
    h = None
    for j, xr in enumerate(xs):
        part = jax.lax.dot_general(
            w1[:, j * ks:(j + 1) * ks], xr[0],
            (((1,), (0,)), ((), ())),
            preferred_element_type=jnp.float32)
        h = part if h is None else h + part
    h = jnp.maximum(h + b1_ref[...].T, 0.0)
    z = jax.lax.dot_general(
        w2_ref[...], h, (((1,), (0,)), ((), ())),
        preferred_element_type=jnp.float32)
    h2 = jnp.maximum(z + b2_ref[...].T, 0.0) * w3_ref[...].T
    o_ref[...] = jnp.sum(h2, axis=0) + b3_ref[0, 0]


@functools.partial(jax.jit, static_argnames=())
def kernel(modes_vec, W1, b1, W2, b2, W3, b3, type_n):
    del type_n
    batch, steps = modes_vec.shape
    blk = min(_LANE_BLK, batch)
    grid = (batch // blk,)
    ks = steps // _NSPLIT

    xt = modes_vec.T
    xt3 = xt.reshape(_NSPLIT, ks, batch)

    def slab(j):
        return pl.BlockSpec((1, ks, blk), lambda i, j=j: (j, 0, i))

    full = lambda i: (0, 0)
    outt = pl.pallas_call(
        _mlp_kernel,
        grid=grid,
        in_specs=[slab(j) for j in range(_NSPLIT)] + [
            pl.BlockSpec(W1.shape, full),
            pl.BlockSpec((1, W1.shape[0]), full),
            pl.BlockSpec(W2.shape, full),
            pl.BlockSpec((1, W2.shape[0]), full),
            pl.BlockSpec(W3.shape, full),
            pl.BlockSpec((1, 1), full),
        ],
        out_specs=pl.BlockSpec((blk,), lambda i: (i,)),
        out_shape=jax.ShapeDtypeStruct((batch,), jnp.float32),
        compiler_params=pltpu.CompilerParams(
            dimension_semantics=("parallel",),
        ),
    )(*([xt3] * _NSPLIT), W1, b1.reshape(1, -1), W2, b2.reshape(1, -1),
      W3, b3.reshape(1, -1))
    return outt.reshape(batch, 1)
